# baseline (speedup 1.0000x reference)
import functools

import jax
import jax.numpy as jnp
from jax import lax
from jax.experimental import pallas as pl
from jax.experimental.pallas import tpu as pltpu
from jax.experimental.pallas import tpu_sc as plsc

N = 10000
E = 160000
N_PAD = 10240
E_PAD = 163840
NW = 32
STEPS = 40
CHUNK = 128
ROWS_PER_TILE = N_PAD // 16
NB = N_PAD // 128


def _make_s_kernel(d):
    mesh = plsc.VectorSubcoreMesh(core_axis_name="c", subcore_axis_name="s")

    @functools.partial(
        pl.kernel,
        out_type=jax.ShapeDtypeStruct((2, N_PAD, d), jnp.float32),
        mesh=mesh,
        scratch_types=[
            pltpu.VMEM((STEPS, CHUNK), jnp.int32),
            pltpu.VMEM((STEPS, CHUNK), jnp.int32),
            pltpu.VMEM((CHUNK,), jnp.int32),
            pltpu.VMEM((CHUNK,), jnp.int32),
            pltpu.VMEM((CHUNK,), jnp.int32),
            pltpu.VMEM((CHUNK,), jnp.int32),
            pltpu.VMEM((CHUNK, d), jnp.float32),
            pltpu.VMEM((CHUNK, d), jnp.float32),
            pltpu.VMEM_SHARED((N_PAD, d), jnp.float32),
            pltpu.SemaphoreType.DMA,
            pltpu.SemaphoreType.DMA,
        ],
        compiler_params=pltpu.CompilerParams(use_tc_tiling_on_sc=False),
    )
    def s_kernel(t_hbm, gat_hbm, sca_hbm, out_hbm, sidx_v, didx_v, dcur_a,
                 dcur_b, dcur_c, dcur_d, rows_a, rows_b, acc_sh, sem_a,
                 sem_b):
        cid = lax.axis_index("c")
        sid = lax.axis_index("s")
        wid = sid * 2 + cid

        pltpu.sync_copy(gat_hbm.at[wid], sidx_v)
        pltpu.sync_copy(sca_hbm.at[wid], didx_v)

        def zero_row(r, carry):
            for c in range(d // 16):
                rows_a[r, pl.ds(c * 16, 16)] = jnp.zeros((16,), jnp.float32)
            return carry

        lax.fori_loop(0, CHUNK, zero_row, 0)
        for k in range(ROWS_PER_TILE // CHUNK):
            pltpu.sync_copy(
                rows_a,
                acc_sh.at[pl.ds(sid * ROWS_PER_TILE + k * CHUNK, CHUNK)])
        plsc.subcore_barrier()

        bufs = (rows_a, rows_b)
        sems = (sem_a, sem_b)
        dcurs = (dcur_a, dcur_b, dcur_c, dcur_d)
        pltpu.async_copy(t_hbm.at[sidx_v.at[0]], rows_a, sem_a)

        def step4(j4, carry):
            j = j4 * 4
            for p in range(4):
                buf, sem = bufs[p % 2], sems[p % 2]
                nbuf, nsem = bufs[(p + 1) % 2], sems[(p + 1) % 2]
                dcur = dcurs[p]
                pltpu.make_async_copy(t_hbm.at[sidx_v.at[j + p]], buf,
                                      sem).wait()
                @pl.when(j + p + 1 < STEPS)
                def _():
                    pltpu.async_copy(t_hbm.at[sidx_v.at[j + p + 1]], nbuf,
                                     nsem)
                for k in range(CHUNK // 16):
                    dcur[pl.ds(k * 16, 16)] = didx_v[j + p,
                                                     pl.ds(k * 16, 16)]
                pltpu.sync_copy(buf, acc_sh.at[dcur], add=True)
            return carry

        lax.fori_loop(0, STEPS // 4, step4, 0)
        plsc.subcore_barrier()
        pltpu.sync_copy(
            acc_sh.at[pl.ds(sid * ROWS_PER_TILE, ROWS_PER_TILE)],
            out_hbm.at[cid, pl.ds(sid * ROWS_PER_TILE, ROWS_PER_TILE)])

    return s_kernel


def _dis_kernel(d0, d1):
    def body(a_ref, b_ref, o_ref):
        deg = a_ref[...] + b_ref[...]
        o_ref[...] = jnp.where(
            deg > 0, 1.0 / jnp.sqrt(jnp.maximum(deg, 1e-12)), 0.0)

    return pl.pallas_call(
        body, out_shape=jax.ShapeDtypeStruct((NB, 128), jnp.float32))(d0, d1)


def _mm3(h, dis_col, W):
    din = h.shape[1]
    dout = W.shape[2]

    def body(h_ref, dis_ref, w0_ref, w1_ref, w2_ref, a_ref, ub_ref, c_ref):
        hb = h_ref[...]
        dv = dis_ref[...]
        a_ref[...] = jnp.dot(hb, w1_ref[...],
                             preferred_element_type=jnp.float32,
                             precision=jax.lax.Precision.HIGHEST)
        ub_ref[...] = dv * jnp.dot(hb, w2_ref[...],
                                   preferred_element_type=jnp.float32,
                             precision=jax.lax.Precision.HIGHEST)
        c_ref[...] = jnp.dot(hb, w0_ref[...] - w2_ref[...],
                             preferred_element_type=jnp.float32,
                             precision=jax.lax.Precision.HIGHEST)

    wspec = pl.BlockSpec((din, dout), lambda i: (0, 0))
    return pl.pallas_call(
        body,
        grid=(NB,),
        in_specs=[
            pl.BlockSpec((128, din), lambda i: (i, 0)),
            pl.BlockSpec((128, 1), lambda i: (i, 0)),
            wspec, wspec, wspec,
        ],
        out_specs=[pl.BlockSpec((128, dout), lambda i: (i, 0))] * 3,
        out_shape=[jax.ShapeDtypeStruct((N_PAD, dout), jnp.float32)] * 3,
    )(h, dis_col, W[0], W[1], W[2])


def _comb1(A, s10, s11, dis_col):
    dout = A.shape[1]

    def body(a_ref, p_ref, q_ref, dis_ref, o_ref):
        dv = dis_ref[...]
        o_ref[...] = dv * (a_ref[...] - 2.0 * dv * (p_ref[...] + q_ref[...]))

    return pl.pallas_call(
        body,
        grid=(NB,),
        in_specs=[pl.BlockSpec((128, dout), lambda i: (i, 0))] * 3
        + [pl.BlockSpec((128, 1), lambda i: (i, 0))],
        out_specs=pl.BlockSpec((128, dout), lambda i: (i, 0)),
        out_shape=jax.ShapeDtypeStruct((N_PAD, dout), jnp.float32),
    )(A, s10, s11, dis_col)


def _final(C, s20, s21, dis_col, b):
    dout = C.shape[1]

    def body(c_ref, p_ref, q_ref, dis_ref, b_ref, h_ref, u_ref):
        dv = dis_ref[...]
        h = jnp.maximum(
            c_ref[...] - dv * (p_ref[...] + q_ref[...]) + b_ref[...], 0.0)
        h_ref[...] = h
        u_ref[...] = dv * h

    return pl.pallas_call(
        body,
        grid=(NB,),
        in_specs=[pl.BlockSpec((128, dout), lambda i: (i, 0))] * 3
        + [pl.BlockSpec((128, 1), lambda i: (i, 0)),
           pl.BlockSpec((1, dout), lambda i: (0, 0))],
        out_specs=[pl.BlockSpec((128, dout), lambda i: (i, 0))] * 2,
        out_shape=[jax.ShapeDtypeStruct((N_PAD, dout), jnp.float32)] * 2,
    )(C, s20, s21, dis_col, b)


def _comb2(s10, s11, dis_col):
    dout = s10.shape[1]

    def body(p_ref, q_ref, dis_ref, t_ref, v_ref):
        dv = dis_ref[...]
        t = -dv * (p_ref[...] + q_ref[...])
        t_ref[...] = t
        v_ref[...] = dv * t

    return pl.pallas_call(
        body,
        grid=(NB,),
        in_specs=[pl.BlockSpec((128, dout), lambda i: (i, 0))] * 2
        + [pl.BlockSpec((128, 1), lambda i: (i, 0))],
        out_specs=[pl.BlockSpec((128, dout), lambda i: (i, 0))] * 2,
        out_shape=[jax.ShapeDtypeStruct((N_PAD, dout), jnp.float32)] * 2,
    )(s10, s11, dis_col)


def _layermm(h, tx1, s20, s21, dis_col, W, b):
    din = h.shape[1]
    dout = W.shape[2]

    def body(h_ref, t1_ref, p_ref, q_ref, dis_ref, w0_ref, w1_ref, w2_ref,
             b_ref, hn_ref, un_ref):
        dv = dis_ref[...]
        tx2p = -dv * (p_ref[...] + q_ref[...])
        acc = jnp.dot(h_ref[...], w0_ref[...] - w2_ref[...],
                      preferred_element_type=jnp.float32,
                             precision=jax.lax.Precision.HIGHEST)
        acc = acc + jnp.dot(t1_ref[...], w1_ref[...],
                            preferred_element_type=jnp.float32,
                             precision=jax.lax.Precision.HIGHEST)
        acc = acc + 2.0 * jnp.dot(tx2p, w2_ref[...],
                                  preferred_element_type=jnp.float32,
                             precision=jax.lax.Precision.HIGHEST)
        hn = jnp.maximum(acc + b_ref[...], 0.0)
        hn_ref[...] = hn
        un_ref[...] = dv * hn

    wspec = pl.BlockSpec((din, dout), lambda i: (0, 0))
    return pl.pallas_call(
        body,
        grid=(NB,),
        in_specs=[pl.BlockSpec((128, din), lambda i: (i, 0))] * 4
        + [pl.BlockSpec((128, 1), lambda i: (i, 0)),
           wspec, wspec, wspec,
           pl.BlockSpec((1, dout), lambda i: (0, 0))],
        out_specs=[pl.BlockSpec((128, dout), lambda i: (i, 0))] * 2,
        out_shape=[jax.ShapeDtypeStruct((N_PAD, dout), jnp.float32)] * 2,
    )(h, tx1, s20, s21, dis_col, W[0], W[1], W[2], b)


def kernel(x, edge_index, Ws, bs):
    src = edge_index[0]
    dst = edge_index[1]
    pad = E_PAD - E
    pad0 = jnp.zeros((pad,), jnp.int32)
    padN = jnp.full((pad,), N, jnp.int32)
    src_g = jnp.concatenate([src, pad0]).reshape(NW, STEPS, CHUNK)
    dst_g = jnp.concatenate([dst, padN]).reshape(NW, STEPS, CHUNK)
    srcs_g = jnp.concatenate([src, padN]).reshape(NW, STEPS, CHUNK)

    x_pad = jnp.pad(x, ((0, N_PAD - N), (0, 0)))
    ones16 = jnp.ones((N_PAD, 16), jnp.float32)

    s16 = _make_s_kernel(16)
    s32 = _make_s_kernel(32)
    s64 = _make_s_kernel(64)

    degp = s16(ones16, src_g, srcs_g)
    dis2d = _dis_kernel(degp[0, :, 0].reshape(NB, 128),
                        degp[1, :, 0].reshape(NB, 128))
    dis_col = dis2d.reshape(N_PAD, 1)

    A, uB, C = _mm3(x_pad, dis_col, Ws[0])
    s1 = s16(uB, src_g, dst_g)
    uq = _comb1(A, s1[0], s1[1], dis_col)
    s2 = s16(uq, src_g, dst_g)
    h, u = _final(C, s2[0], s2[1], dis_col, bs[0].reshape(1, -1))

    for l, sk in ((1, s16), (2, s32), (3, s64)):
        s1 = sk(u, src_g, dst_g)
        tx1, v = _comb2(s1[0], s1[1], dis_col)
        s2 = sk(v, src_g, dst_g)
        h, u = _layermm(h, tx1, s2[0], s2[1], dis_col, Ws[l],
                        bs[l].reshape(1, -1))

    W5 = jnp.pad(Ws[4], ((0, 0), (0, 0), (0, 11)))
    b5 = jnp.pad(bs[4], (0, 11)).reshape(1, -1)
    A, uB, C = _mm3(h, dis_col, W5)
    s1 = s16(uB, src_g, dst_g)
    uq = _comb1(A, s1[0], s1[1], dis_col)
    s2 = s16(uq, src_g, dst_g)
    out, _ = _final(C, s2[0], s2[1], dis_col, b5)
    return out[:N, :5]


# speedup vs baseline: 1.3496x; 1.3496x over previous
import functools

import jax
import jax.numpy as jnp
from jax import lax
from jax.experimental import pallas as pl
from jax.experimental.pallas import tpu as pltpu
from jax.experimental.pallas import tpu_sc as plsc

N = 10000
E = 160000
N_PAD = 10240
E_PAD = 163840
NW = 32
STEPS = 40
CHUNK = 128
ROWS_PER_TILE = N_PAD // 16
NB = N_PAD // 128
NBR = (N + 127) // 128


def _make_s_kernel(d):
    mesh = plsc.VectorSubcoreMesh(core_axis_name="c", subcore_axis_name="s")

    @functools.partial(
        pl.kernel,
        out_type=jax.ShapeDtypeStruct((2, N_PAD, d), jnp.float32),
        mesh=mesh,
        scratch_types=[
            pltpu.VMEM((STEPS, CHUNK), jnp.int32),
            pltpu.VMEM((STEPS, CHUNK), jnp.int32),
            pltpu.VMEM((CHUNK,), jnp.int32),
            pltpu.VMEM((CHUNK,), jnp.int32),
            pltpu.VMEM((CHUNK,), jnp.int32),
            pltpu.VMEM((CHUNK,), jnp.int32),
            pltpu.VMEM((CHUNK, d), jnp.float32),
            pltpu.VMEM((CHUNK, d), jnp.float32),
            pltpu.VMEM_SHARED((N_PAD, d), jnp.float32),
            pltpu.SemaphoreType.DMA,
            pltpu.SemaphoreType.DMA,
        ],
        compiler_params=pltpu.CompilerParams(use_tc_tiling_on_sc=False),
    )
    def s_kernel(t_hbm, gat_hbm, sca_hbm, out_hbm, sidx_v, didx_v, dcur_a,
                 dcur_b, dcur_c, dcur_d, rows_a, rows_b, acc_sh, sem_a,
                 sem_b):
        cid = lax.axis_index("c")
        sid = lax.axis_index("s")
        wid = sid * 2 + cid

        pltpu.sync_copy(gat_hbm.at[wid], sidx_v)
        pltpu.sync_copy(sca_hbm.at[wid], didx_v)

        def zero_row(r, carry):
            for c in range(d // 16):
                rows_a[r, pl.ds(c * 16, 16)] = jnp.zeros((16,), jnp.float32)
            return carry

        lax.fori_loop(0, CHUNK, zero_row, 0)
        for k in range(ROWS_PER_TILE // CHUNK):
            pltpu.sync_copy(
                rows_a,
                acc_sh.at[pl.ds(sid * ROWS_PER_TILE + k * CHUNK, CHUNK)])
        plsc.subcore_barrier()

        bufs = (rows_a, rows_b)
        sems = (sem_a, sem_b)
        dcurs = (dcur_a, dcur_b, dcur_c, dcur_d)
        pltpu.async_copy(t_hbm.at[sidx_v.at[0]], rows_a, sem_a)

        def step4(j4, carry):
            j = j4 * 4
            for p in range(4):
                buf, sem = bufs[p % 2], sems[p % 2]
                nbuf, nsem = bufs[(p + 1) % 2], sems[(p + 1) % 2]
                dcur = dcurs[p]
                pltpu.make_async_copy(t_hbm.at[sidx_v.at[j + p]], buf,
                                      sem).wait()
                @pl.when(j + p + 1 < STEPS)
                def _():
                    pltpu.async_copy(t_hbm.at[sidx_v.at[j + p + 1]], nbuf,
                                     nsem)
                for k in range(CHUNK // 16):
                    dcur[pl.ds(k * 16, 16)] = didx_v[j + p,
                                                     pl.ds(k * 16, 16)]
                pltpu.sync_copy(buf, acc_sh.at[dcur], add=True)
            return carry

        lax.fori_loop(0, STEPS // 4, step4, 0)
        plsc.subcore_barrier()
        pltpu.sync_copy(
            acc_sh.at[pl.ds(sid * ROWS_PER_TILE, ROWS_PER_TILE)],
            out_hbm.at[cid, pl.ds(sid * ROWS_PER_TILE, ROWS_PER_TILE)])

    return s_kernel


def _dis_kernel(d0, d1):
    def body(a_ref, b_ref, o_ref):
        deg = a_ref[...] + b_ref[...]
        o_ref[...] = jnp.where(
            deg > 0, 1.0 / jnp.sqrt(jnp.maximum(deg, 1e-12)), 0.0)

    return pl.pallas_call(
        body, out_shape=jax.ShapeDtypeStruct((NB, 128), jnp.float32))(d0, d1)


def _mm3(h, dis_col, W):
    din = h.shape[1]
    dout = W.shape[2]

    def body(h_ref, dis_ref, w0_ref, w1_ref, w2_ref, a_ref, ub_ref, c_ref):
        hb = h_ref[...]
        dv = dis_ref[...]
        a_ref[...] = jnp.dot(hb, w1_ref[...],
                             preferred_element_type=jnp.float32,
                             precision=jax.lax.Precision.HIGHEST)
        ub_ref[...] = dv * jnp.dot(hb, w2_ref[...],
                                   preferred_element_type=jnp.float32,
                             precision=jax.lax.Precision.HIGHEST)
        c_ref[...] = jnp.dot(hb, w0_ref[...] - w2_ref[...],
                             preferred_element_type=jnp.float32,
                             precision=jax.lax.Precision.HIGHEST)

    wspec = pl.BlockSpec((din, dout), lambda i: (0, 0))
    return pl.pallas_call(
        body,
        grid=(NBR,),
        in_specs=[
            pl.BlockSpec((128, din), lambda i: (i, 0)),
            pl.BlockSpec((128, 1), lambda i: (i, 0)),
            wspec, wspec, wspec,
        ],
        out_specs=[pl.BlockSpec((128, dout), lambda i: (i, 0))] * 3,
        out_shape=[jax.ShapeDtypeStruct((N, dout), jnp.float32)] * 3,
    )(h, dis_col, W[0], W[1], W[2])


def _comb1(A, s10, s11, dis_col):
    dout = A.shape[1]

    def body(a_ref, p_ref, q_ref, dis_ref, o_ref):
        dv = dis_ref[...]
        o_ref[...] = dv * (a_ref[...] - 2.0 * dv * (p_ref[...] + q_ref[...]))

    return pl.pallas_call(
        body,
        grid=(NBR,),
        in_specs=[pl.BlockSpec((128, dout), lambda i: (i, 0))] * 3
        + [pl.BlockSpec((128, 1), lambda i: (i, 0))],
        out_specs=pl.BlockSpec((128, dout), lambda i: (i, 0)),
        out_shape=jax.ShapeDtypeStruct((N, dout), jnp.float32),
    )(A, s10, s11, dis_col)


def _final(C, s20, s21, dis_col, b):
    dout = C.shape[1]

    def body(c_ref, p_ref, q_ref, dis_ref, b_ref, h_ref, u_ref):
        dv = dis_ref[...]
        h = jnp.maximum(
            c_ref[...] - dv * (p_ref[...] + q_ref[...]) + b_ref[...], 0.0)
        h_ref[...] = h
        u_ref[...] = dv * h

    return pl.pallas_call(
        body,
        grid=(NBR,),
        in_specs=[pl.BlockSpec((128, dout), lambda i: (i, 0))] * 3
        + [pl.BlockSpec((128, 1), lambda i: (i, 0)),
           pl.BlockSpec((1, dout), lambda i: (0, 0))],
        out_specs=[pl.BlockSpec((128, dout), lambda i: (i, 0))] * 2,
        out_shape=[jax.ShapeDtypeStruct((N, dout), jnp.float32)] * 2,
    )(C, s20, s21, dis_col, b)


def _comb2(s10, s11, dis_col):
    dout = s10.shape[1]

    def body(p_ref, q_ref, dis_ref, t_ref, v_ref):
        dv = dis_ref[...]
        t = -dv * (p_ref[...] + q_ref[...])
        t_ref[...] = t
        v_ref[...] = dv * t

    return pl.pallas_call(
        body,
        grid=(NBR,),
        in_specs=[pl.BlockSpec((128, dout), lambda i: (i, 0))] * 2
        + [pl.BlockSpec((128, 1), lambda i: (i, 0))],
        out_specs=[pl.BlockSpec((128, dout), lambda i: (i, 0))] * 2,
        out_shape=[jax.ShapeDtypeStruct((N, dout), jnp.float32)] * 2,
    )(s10, s11, dis_col)


def _layermm(h, tx1, s20, s21, dis_col, W, b):
    din = h.shape[1]
    dout = W.shape[2]

    def body(h_ref, t1_ref, p_ref, q_ref, dis_ref, w0_ref, w1_ref, w2_ref,
             b_ref, hn_ref, un_ref):
        dv = dis_ref[...]
        tx2p = -dv * (p_ref[...] + q_ref[...])
        acc = jnp.dot(h_ref[...], w0_ref[...] - w2_ref[...],
                      preferred_element_type=jnp.float32,
                             precision=jax.lax.Precision.HIGHEST)
        acc = acc + jnp.dot(t1_ref[...], w1_ref[...],
                            preferred_element_type=jnp.float32,
                             precision=jax.lax.Precision.HIGHEST)
        acc = acc + 2.0 * jnp.dot(tx2p, w2_ref[...],
                                  preferred_element_type=jnp.float32,
                             precision=jax.lax.Precision.HIGHEST)
        hn = jnp.maximum(acc + b_ref[...], 0.0)
        hn_ref[...] = hn
        un_ref[...] = dv * hn

    wspec = pl.BlockSpec((din, dout), lambda i: (0, 0))
    return pl.pallas_call(
        body,
        grid=(NBR,),
        in_specs=[pl.BlockSpec((128, din), lambda i: (i, 0))] * 4
        + [pl.BlockSpec((128, 1), lambda i: (i, 0)),
           wspec, wspec, wspec,
           pl.BlockSpec((1, dout), lambda i: (0, 0))],
        out_specs=[pl.BlockSpec((128, dout), lambda i: (i, 0))] * 2,
        out_shape=[jax.ShapeDtypeStruct((N, dout), jnp.float32)] * 2,
    )(h, tx1, s20, s21, dis_col, W[0], W[1], W[2], b)


def kernel(x, edge_index, Ws, bs):
    src = edge_index[0]
    dst = edge_index[1]
    pad = E_PAD - E
    j = jnp.arange(pad, dtype=jnp.int32)
    pad_gat = (j * 64) % 9984
    pad_sca = N + (j % (N_PAD - N))
    src_g = jnp.concatenate([src, pad_gat]).reshape(NW, STEPS, CHUNK)
    dst_g = jnp.concatenate([dst, pad_sca]).reshape(NW, STEPS, CHUNK)
    srcs_g = jnp.concatenate([src, pad_sca]).reshape(NW, STEPS, CHUNK)

    ones16 = jnp.ones((N, 16), jnp.float32)

    s16 = _make_s_kernel(16)
    s32 = _make_s_kernel(32)
    s64 = _make_s_kernel(64)

    degp = s16(ones16, src_g, srcs_g)
    dis2d = _dis_kernel(degp[0, :, 0].reshape(NB, 128),
                        degp[1, :, 0].reshape(NB, 128))
    dis_col = dis2d.reshape(N_PAD, 1)

    A, uB, C = _mm3(x, dis_col, Ws[0])
    s1 = s16(uB, src_g, dst_g)
    uq = _comb1(A, s1[0], s1[1], dis_col)
    s2 = s16(uq, src_g, dst_g)
    h, u = _final(C, s2[0], s2[1], dis_col, bs[0].reshape(1, -1))

    for l, sk in ((1, s16), (2, s32), (3, s64)):
        s1 = sk(u, src_g, dst_g)
        tx1, v = _comb2(s1[0], s1[1], dis_col)
        s2 = sk(v, src_g, dst_g)
        h, u = _layermm(h, tx1, s2[0], s2[1], dis_col, Ws[l],
                        bs[l].reshape(1, -1))

    W5 = jnp.pad(Ws[4], ((0, 0), (0, 0), (0, 11)))
    b5 = jnp.pad(bs[4], (0, 11)).reshape(1, -1)
    A, uB, C = _mm3(h, dis_col, W5)
    s1 = s16(uB, src_g, dst_g)
    uq = _comb1(A, s1[0], s1[1], dis_col)
    s2 = s16(uq, src_g, dst_g)
    out, _ = _final(C, s2[0], s2[1], dis_col, b5)
    return out[:, :5]


# speedup vs baseline: 1.8173x; 1.3465x over previous
import functools

import jax
import jax.numpy as jnp
from jax import lax
from jax.experimental import pallas as pl
from jax.experimental.pallas import tpu as pltpu
from jax.experimental.pallas import tpu_sc as plsc

N = 10000
E = 160000
N_PAD = 10240
E_PAD = 163840
NW = 32
STEPS = 40
CHUNK = 128
ROWS_PER_TILE = N_PAD // 16
NB = N_PAD // 128
NBR = (N + 127) // 128


def _make_s_kernel(d):
    mesh = plsc.VectorSubcoreMesh(core_axis_name="c", subcore_axis_name="s")

    @functools.partial(
        pl.kernel,
        out_type=jax.ShapeDtypeStruct((2, N_PAD, d), jnp.float32),
        mesh=mesh,
        scratch_types=[
            pltpu.VMEM((STEPS, CHUNK), jnp.int32),
            pltpu.VMEM((STEPS, CHUNK), jnp.int32),
            pltpu.VMEM((CHUNK,), jnp.int32),
            pltpu.VMEM((CHUNK,), jnp.int32),
            pltpu.VMEM((CHUNK,), jnp.int32),
            pltpu.VMEM((CHUNK,), jnp.int32),
            pltpu.VMEM((CHUNK, d), jnp.float32),
            pltpu.VMEM((CHUNK, d), jnp.float32),
            pltpu.VMEM_SHARED((N_PAD, d), jnp.float32),
            pltpu.SemaphoreType.DMA,
            pltpu.SemaphoreType.DMA,
        ],
        compiler_params=pltpu.CompilerParams(use_tc_tiling_on_sc=False),
    )
    def s_kernel(t_hbm, gat_hbm, sca_hbm, out_hbm, sidx_v, didx_v, dcur_a,
                 dcur_b, dcur_c, dcur_d, rows_a, rows_b, acc_sh, sem_a,
                 sem_b):
        cid = lax.axis_index("c")
        sid = lax.axis_index("s")
        wid = sid * 2 + cid

        pltpu.sync_copy(gat_hbm.at[wid], sidx_v)
        pltpu.sync_copy(sca_hbm.at[wid], didx_v)

        def zero_row(r, carry):
            for c in range(d // 16):
                rows_a[r, pl.ds(c * 16, 16)] = jnp.zeros((16,), jnp.float32)
            return carry

        lax.fori_loop(0, CHUNK, zero_row, 0)
        for k in range(ROWS_PER_TILE // CHUNK):
            pltpu.sync_copy(
                rows_a,
                acc_sh.at[pl.ds(sid * ROWS_PER_TILE + k * CHUNK, CHUNK)])
        plsc.subcore_barrier()

        bufs = (rows_a, rows_b)
        sems = (sem_a, sem_b)
        dcurs = (dcur_a, dcur_b, dcur_c, dcur_d)
        pltpu.async_copy(t_hbm.at[sidx_v.at[0]], rows_a, sem_a)

        def step4(j4, carry):
            j = j4 * 4
            for p in range(4):
                buf, sem = bufs[p % 2], sems[p % 2]
                nbuf, nsem = bufs[(p + 1) % 2], sems[(p + 1) % 2]
                dcur = dcurs[p]
                pltpu.make_async_copy(t_hbm.at[sidx_v.at[j + p]], buf,
                                      sem).wait()
                @pl.when(j + p + 1 < STEPS)
                def _():
                    pltpu.async_copy(t_hbm.at[sidx_v.at[j + p + 1]], nbuf,
                                     nsem)
                for k in range(CHUNK // 16):
                    dcur[pl.ds(k * 16, 16)] = didx_v[j + p,
                                                     pl.ds(k * 16, 16)]
                pltpu.sync_copy(buf, acc_sh.at[dcur], add=True)
            return carry

        lax.fori_loop(0, STEPS // 4, step4, 0)
        plsc.subcore_barrier()
        pltpu.sync_copy(
            acc_sh.at[pl.ds(sid * ROWS_PER_TILE, ROWS_PER_TILE)],
            out_hbm.at[cid, pl.ds(sid * ROWS_PER_TILE, ROWS_PER_TILE)])

    return s_kernel


def _make_fused_kernel(d, mm_first):
    S2 = 2 * STEPS
    RT = 624
    mesh = plsc.VectorSubcoreMesh(core_axis_name="c", subcore_axis_name="s")
    n_out = 1 if mm_first else 2
    outs = [jax.ShapeDtypeStruct((N_PAD, d), jnp.float32)] * n_out

    def body(*refs):
        if mm_first:
            (u_hbm, ua_hbm, gat_hbm, sca_hbm, nd2_hbm, s2_hbm,
             sidx_v, didx_v, dcur_a, dcur_b, dcur_c, dcur_d,
             rows_a, rows_b, scl_a, scl_b, scl_c, nd2_v, u_sh, acc_sh,
             sem_a, sem_b) = refs
            s1_hbm = None
        else:
            (u_hbm, gat_hbm, sca_hbm, nd2_hbm, s1_hbm, s2_hbm,
             sidx_v, didx_v, dcur_a, dcur_b, dcur_c, dcur_d,
             rows_a, rows_b, scl_a, scl_b, scl_c, nd2_v, u_sh, acc_sh,
             sem_a, sem_b) = refs
        cid = lax.axis_index("c")
        sid = lax.axis_index("s")

        pltpu.sync_copy(gat_hbm.at[2 * sid], sidx_v.at[pl.ds(0, STEPS)])
        pltpu.sync_copy(gat_hbm.at[2 * sid + 1],
                        sidx_v.at[pl.ds(STEPS, STEPS)])
        pltpu.sync_copy(sca_hbm.at[2 * sid], didx_v.at[pl.ds(0, STEPS)])
        pltpu.sync_copy(sca_hbm.at[2 * sid + 1],
                        didx_v.at[pl.ds(STEPS, STEPS)])
        pltpu.sync_copy(nd2_hbm.at[pl.ds(sid * RT, RT)],
                        nd2_v.at[pl.ds(0, RT)])

        pltpu.sync_copy(u_hbm.at[pl.ds(sid * RT, RT)],
                        u_sh.at[pl.ds(sid * RT, RT)])

        @pl.when(sid == 15)
        def _():
            pltpu.sync_copy(nd2_hbm.at[pl.ds(16 * RT, 16)],
                            nd2_v.at[pl.ds(RT, 16)])
            pltpu.sync_copy(u_hbm.at[pl.ds(16 * RT, 16)],
                            u_sh.at[pl.ds(16 * RT, 16)])

        def zero_row(r, carry):
            for c in range(d // 16):
                rows_a[r, pl.ds(c * 16, 16)] = jnp.zeros((16,), jnp.float32)
            return carry

        lax.fori_loop(0, CHUNK, zero_row, 0)
        for k in range(ROWS_PER_TILE // CHUNK):
            pltpu.sync_copy(
                rows_a,
                acc_sh.at[pl.ds(sid * ROWS_PER_TILE + k * CHUNK, CHUNK)])
        plsc.subcore_barrier()

        bufs = (rows_a, rows_b)
        sems = (sem_a, sem_b)
        dcurs = (dcur_a, dcur_b, dcur_c, dcur_d)

        def edge_pass():
            pltpu.async_copy(u_sh.at[sidx_v.at[0]], rows_a, sem_a)

            def step4(j4, carry):
                j = j4 * 4
                for p in range(4):
                    buf, sem = bufs[p % 2], sems[p % 2]
                    nbuf, nsem = bufs[(p + 1) % 2], sems[(p + 1) % 2]
                    dcur = dcurs[p]
                    pltpu.make_async_copy(u_sh.at[sidx_v.at[j + p]], buf,
                                          sem).wait()
                    @pl.when(j + p + 1 < S2)
                    def _():
                        pltpu.async_copy(u_sh.at[sidx_v.at[j + p + 1]],
                                         nbuf, nsem)
                    for k in range(CHUNK // 16):
                        dcur[pl.ds(k * 16, 16)] = didx_v[j + p,
                                                         pl.ds(k * 16, 16)]
                    pltpu.sync_copy(buf, acc_sh.at[dcur], add=True)
                return carry

            lax.fori_loop(0, S2 // 4, step4, 0)

        edge_pass()
        plsc.subcore_barrier()

        if not mm_first:
            pltpu.sync_copy(
                acc_sh.at[pl.ds(cid * 5120 + sid * 320, 320)],
                s1_hbm.at[pl.ds(cid * 5120 + sid * 320, 320)])

        def scale_chunk(gbase, goff, nrows, sbuf):
            pltpu.sync_copy(acc_sh.at[pl.ds(gbase, nrows)],
                            sbuf.at[pl.ds(0, nrows)])
            if mm_first:
                pltpu.sync_copy(ua_hbm.at[pl.ds(gbase, nrows)],
                                rows_a.at[pl.ds(0, nrows)])

            def scale_group(g, carry):
                nd_vec = nd2_v[pl.ds(goff + g * 16, 16)]
                for k in range(16):
                    r = g * 16 + k
                    s = nd_vec[k]
                    for c in range(d // 16):
                        cur = sbuf[r, pl.ds(c * 16, 16)]
                        if mm_first:
                            cur = (rows_a[r, pl.ds(c * 16, 16)]
                                   + 2.0 * s * cur)
                        else:
                            cur = s * cur
                        sbuf[r, pl.ds(c * 16, 16)] = cur
                return carry

            lax.fori_loop(0, nrows // 16, scale_group, 0)
            pltpu.sync_copy(sbuf.at[pl.ds(0, nrows)],
                            u_sh.at[pl.ds(gbase, nrows)])

        sbufs = (scl_a, scl_b, scl_c)
        for i5, (c_off, c_sz) in enumerate(((0, 128), (128, 128),
                                            (256, 128), (384, 128),
                                            (512, 112))):
            scale_chunk(sid * RT + c_off, c_off, c_sz, sbufs[i5 % 3])

        @pl.when(sid == 15)
        def _():
            scale_chunk(16 * RT, RT, 16, scl_c)
        plsc.subcore_barrier()

        lax.fori_loop(0, CHUNK, zero_row, 0)
        for k in range(ROWS_PER_TILE // CHUNK):
            pltpu.sync_copy(
                rows_a,
                acc_sh.at[pl.ds(sid * ROWS_PER_TILE + k * CHUNK, CHUNK)])
        plsc.subcore_barrier()

        edge_pass()
        plsc.subcore_barrier()
        pltpu.sync_copy(
            acc_sh.at[pl.ds(cid * 5120 + sid * 320, 320)],
            s2_hbm.at[pl.ds(cid * 5120 + sid * 320, 320)])

    scratch = [
        pltpu.VMEM((S2, CHUNK), jnp.int32),
        pltpu.VMEM((S2, CHUNK), jnp.int32),
        pltpu.VMEM((CHUNK,), jnp.int32),
        pltpu.VMEM((CHUNK,), jnp.int32),
        pltpu.VMEM((CHUNK,), jnp.int32),
        pltpu.VMEM((CHUNK,), jnp.int32),
        pltpu.VMEM((CHUNK, d), jnp.float32),
        pltpu.VMEM((CHUNK, d), jnp.float32),
        pltpu.VMEM((CHUNK, d), jnp.float32),
        pltpu.VMEM((CHUNK, d), jnp.float32),
        pltpu.VMEM((CHUNK, d), jnp.float32),
        pltpu.VMEM((RT + 16,), jnp.float32),
        pltpu.VMEM_SHARED((N, d), jnp.float32),
        pltpu.VMEM_SHARED((N_PAD, d), jnp.float32),
        pltpu.SemaphoreType.DMA,
        pltpu.SemaphoreType.DMA,
    ]
    kern = functools.partial(
        pl.kernel,
        out_type=outs[0] if n_out == 1 else tuple(outs),
        mesh=mesh,
        scratch_types=scratch,
        compiler_params=pltpu.CompilerParams(use_tc_tiling_on_sc=False),
    )(body)
    return kern


def _dis_kernel(d0, d1):
    def body(a_ref, b_ref, o_ref, n_ref):
        deg = a_ref[...] + b_ref[...]
        dis = jnp.where(
            deg > 0, 1.0 / jnp.sqrt(jnp.maximum(deg, 1e-12)), 0.0)
        o_ref[...] = dis
        n_ref[...] = -(dis * dis)

    return pl.pallas_call(
        body, out_shape=[jax.ShapeDtypeStruct((NB, 128), jnp.float32)] * 2,
    )(d0, d1)


def _mm3(h, dis_col, W):
    din = h.shape[1]
    dout = W.shape[2]

    def body(h_ref, dis_ref, w0_ref, w1_ref, w2_ref, a_ref, ub_ref, c_ref):
        hb = h_ref[...]
        dv = dis_ref[...]
        a_ref[...] = dv * jnp.dot(hb, w1_ref[...],
                                  preferred_element_type=jnp.float32,
                                  precision=jax.lax.Precision.HIGHEST)
        ub_ref[...] = dv * jnp.dot(hb, w2_ref[...],
                                   preferred_element_type=jnp.float32,
                             precision=jax.lax.Precision.HIGHEST)
        c_ref[...] = jnp.dot(hb, w0_ref[...] - w2_ref[...],
                             preferred_element_type=jnp.float32,
                             precision=jax.lax.Precision.HIGHEST)

    wspec = pl.BlockSpec((din, dout), lambda i: (0, 0))
    return pl.pallas_call(
        body,
        grid=(NBR,),
        in_specs=[
            pl.BlockSpec((128, din), lambda i: (i, 0)),
            pl.BlockSpec((128, 1), lambda i: (i, 0)),
            wspec, wspec, wspec,
        ],
        out_specs=[pl.BlockSpec((128, dout), lambda i: (i, 0))] * 3,
        out_shape=[jax.ShapeDtypeStruct((N, dout), jnp.float32)] * 3,
    )(h, dis_col, W[0], W[1], W[2])


def _comb1(A, s10, s11, dis_col):
    dout = A.shape[1]

    def body(a_ref, p_ref, q_ref, dis_ref, o_ref):
        dv = dis_ref[...]
        o_ref[...] = dv * (a_ref[...] - 2.0 * dv * (p_ref[...] + q_ref[...]))

    return pl.pallas_call(
        body,
        grid=(NBR,),
        in_specs=[pl.BlockSpec((128, dout), lambda i: (i, 0))] * 3
        + [pl.BlockSpec((128, 1), lambda i: (i, 0))],
        out_specs=pl.BlockSpec((128, dout), lambda i: (i, 0)),
        out_shape=jax.ShapeDtypeStruct((N, dout), jnp.float32),
    )(A, s10, s11, dis_col)


def _final(C, s2, dis_col, b):
    dout = C.shape[1]

    def body(c_ref, p_ref, dis_ref, b_ref, h_ref, u_ref):
        dv = dis_ref[...]
        h = jnp.maximum(c_ref[...] - dv * p_ref[...] + b_ref[...], 0.0)
        h_ref[...] = h
        u_ref[...] = dv * h

    return pl.pallas_call(
        body,
        grid=(NBR,),
        in_specs=[pl.BlockSpec((128, dout), lambda i: (i, 0))] * 2
        + [pl.BlockSpec((128, 1), lambda i: (i, 0)),
           pl.BlockSpec((1, dout), lambda i: (0, 0))],
        out_specs=[pl.BlockSpec((128, dout), lambda i: (i, 0))] * 2,
        out_shape=[jax.ShapeDtypeStruct((N, dout), jnp.float32)] * 2,
    )(C, s2, dis_col, b)


def _layermm(h, s1, s2, dis_col, W, b):
    din = h.shape[1]
    dout = W.shape[2]

    def body(h_ref, s1_ref, s2_ref, dis_ref, w0_ref, w1_ref, w2_ref,
             b_ref, hn_ref, un_ref):
        dv = dis_ref[...]
        tx1 = -dv * s1_ref[...]
        tx2p = -dv * s2_ref[...]
        acc = jnp.dot(h_ref[...], w0_ref[...] - w2_ref[...],
                      preferred_element_type=jnp.float32,
                      precision=jax.lax.Precision.HIGHEST)
        acc = acc + jnp.dot(tx1, w1_ref[...],
                            preferred_element_type=jnp.float32,
                            precision=jax.lax.Precision.HIGHEST)
        acc = acc + 2.0 * jnp.dot(tx2p, w2_ref[...],
                                  preferred_element_type=jnp.float32,
                                  precision=jax.lax.Precision.HIGHEST)
        hn = jnp.maximum(acc + b_ref[...], 0.0)
        hn_ref[...] = hn
        un_ref[...] = dv * hn

    wspec = pl.BlockSpec((din, dout), lambda i: (0, 0))
    return pl.pallas_call(
        body,
        grid=(NBR,),
        in_specs=[pl.BlockSpec((128, din), lambda i: (i, 0))] * 3
        + [pl.BlockSpec((128, 1), lambda i: (i, 0)),
           wspec, wspec, wspec,
           pl.BlockSpec((1, dout), lambda i: (0, 0))],
        out_specs=[pl.BlockSpec((128, dout), lambda i: (i, 0))] * 2,
        out_shape=[jax.ShapeDtypeStruct((N, dout), jnp.float32)] * 2,
    )(h, s1, s2, dis_col, W[0], W[1], W[2], b)


def _comb2(s10, s11, dis_col):
    dout = s10.shape[1]

    def body(p_ref, q_ref, dis_ref, t_ref, v_ref):
        dv = dis_ref[...]
        t = -dv * (p_ref[...] + q_ref[...])
        t_ref[...] = t
        v_ref[...] = dv * t

    return pl.pallas_call(
        body,
        grid=(NBR,),
        in_specs=[pl.BlockSpec((128, dout), lambda i: (i, 0))] * 2
        + [pl.BlockSpec((128, 1), lambda i: (i, 0))],
        out_specs=[pl.BlockSpec((128, dout), lambda i: (i, 0))] * 2,
        out_shape=[jax.ShapeDtypeStruct((N, dout), jnp.float32)] * 2,
    )(s10, s11, dis_col)


def _layermm_p(h, tx1, s20, s21, dis_col, W, b):
    din = h.shape[1]
    dout = W.shape[2]

    def body(h_ref, t1_ref, p_ref, q_ref, dis_ref, w0_ref, w1_ref, w2_ref,
             b_ref, hn_ref, un_ref):
        dv = dis_ref[...]
        tx2p = -dv * (p_ref[...] + q_ref[...])
        acc = jnp.dot(h_ref[...], w0_ref[...] - w2_ref[...],
                      preferred_element_type=jnp.float32,
                             precision=jax.lax.Precision.HIGHEST)
        acc = acc + jnp.dot(t1_ref[...], w1_ref[...],
                            preferred_element_type=jnp.float32,
                             precision=jax.lax.Precision.HIGHEST)
        acc = acc + 2.0 * jnp.dot(tx2p, w2_ref[...],
                                  preferred_element_type=jnp.float32,
                             precision=jax.lax.Precision.HIGHEST)
        hn = jnp.maximum(acc + b_ref[...], 0.0)
        hn_ref[...] = hn
        un_ref[...] = dv * hn

    wspec = pl.BlockSpec((din, dout), lambda i: (0, 0))
    return pl.pallas_call(
        body,
        grid=(NBR,),
        in_specs=[pl.BlockSpec((128, din), lambda i: (i, 0))] * 4
        + [pl.BlockSpec((128, 1), lambda i: (i, 0)),
           wspec, wspec, wspec,
           pl.BlockSpec((1, dout), lambda i: (0, 0))],
        out_specs=[pl.BlockSpec((128, dout), lambda i: (i, 0))] * 2,
        out_shape=[jax.ShapeDtypeStruct((N, dout), jnp.float32)] * 2,
    )(h, tx1, s20, s21, dis_col, W[0], W[1], W[2], b)


def kernel(x, edge_index, Ws, bs):
    src = edge_index[0]
    dst = edge_index[1]
    pad = E_PAD - E
    j = jnp.arange(pad, dtype=jnp.int32)
    pad_gat = (j * 64) % 9984
    pad_sca = N + (j % (N_PAD - N))
    src_g = jnp.concatenate([src, pad_gat]).reshape(NW, STEPS, CHUNK)
    dst_g = jnp.concatenate([dst, pad_sca]).reshape(NW, STEPS, CHUNK)
    srcs_g = jnp.concatenate([src, pad_sca]).reshape(NW, STEPS, CHUNK)

    ones16 = jnp.ones((N, 16), jnp.float32)

    s16 = _make_s_kernel(16)
    f16m = _make_fused_kernel(16, mm_first=True)
    f16p = _make_fused_kernel(16, mm_first=False)
    f32p = _make_fused_kernel(32, mm_first=False)

    degp = s16(ones16, src_g, srcs_g)
    dis2d, nd2_2d = _dis_kernel(degp[0, :, 0].reshape(NB, 128),
                                degp[1, :, 0].reshape(NB, 128))
    dis_col = dis2d.reshape(N_PAD, 1)
    nd2 = nd2_2d.reshape(N_PAD)

    uA, uB, C = _mm3(x, dis_col, Ws[0])
    s2 = f16m(uB, uA, src_g, dst_g, nd2)
    h, u = _final(C, s2, dis_col, bs[0].reshape(1, -1))

    for l, fk in ((1, f16p), (2, f32p)):
        s1, s2 = fk(u, src_g, dst_g, nd2)
        h, u = _layermm(h, s1, s2, dis_col, Ws[l], bs[l].reshape(1, -1))

    s64 = _make_s_kernel(64)
    s1p = s64(u, src_g, dst_g)
    tx1, v = _comb2(s1p[0], s1p[1], dis_col)
    s2p = s64(v, src_g, dst_g)
    h, u = _layermm_p(h, tx1, s2p[0], s2p[1], dis_col, Ws[3],
                      bs[3].reshape(1, -1))

    W5 = jnp.pad(Ws[4], ((0, 0), (0, 0), (0, 11)))
    b5 = jnp.pad(bs[4], (0, 11)).reshape(1, -1)
    uA, uB, C = _mm3(h, dis_col, W5)
    s2 = f16m(uB, uA, src_g, dst_g, nd2)
    out, _ = _final(C, s2, dis_col, b5)
    return out[:, :5]
